# baseline (speedup 1.0000x reference)
import jax
import jax.numpy as jnp
from jax import lax
from jax.experimental import pallas as pl
from jax.experimental.pallas import tpu as pltpu
from jax.experimental.pallas import tpu_sc as plsc

_ROWS = 4096
_COLS = 512
_N = _ROWS * _COLS

_TC_ROWS = 0
_SC_ROWS = _ROWS - _TC_ROWS

_NW = 32
_ROWS_W = _SC_ROWS // _NW
_CH_ROWS = 16
_NCH = _ROWS_W // _CH_ROWS
_UNROLL = 8
_VSTEPS = _CH_ROWS * (_COLS // 16) // _UNROLL

_TC_BLK = 512

_C = (
    6.944574454159635e-05,
    0.996261948233795,
    -0.46644243862757173,
    0.21866548366223043,
    -0.055459313742085355,
)


def _sc_body(x_hbm, y_hbm, out_hbm, xb0, yb0, xb1, yb1, accv):
    wid = lax.axis_index("s") * 2 + lax.axis_index("c")
    base_row = _TC_ROWS + wid * _ROWS_W
    zero = jnp.zeros((16,), jnp.float32)
    bufs = ((xb0, yb0, None), (xb1, yb1, None))

    def chunk(xb, yb, carry):
        def vstep(vi, carry2):
            t, f, c = carry2
            r = vi // 4
            cb = (vi % 4) * 128
            for u in range(_UNROLL):
                x = xb[r, pl.ds(cb + u * 16, 16)]
                y = yb[r, pl.ds(cb + u * 16, 16)]
                xy = x * y
                w = jnp.exp(jnp.minimum(x, -x))
                l1p = ((((_C[4] * w + _C[3]) * w + _C[2]) * w + _C[1]) * w
                       + _C[0])
                per = jnp.maximum(x, 0.0) - xy + l1p
                t = t + per
                f = f + per * y
                c = c + y
            return (t, f, c)

        return lax.fori_loop(0, _VSTEPS, vstep, carry)

    carry = (zero, zero, zero)
    for ci in range(_NCH):
        xb, yb, _ = bufs[ci & 1]
        r0 = base_row + ci * _CH_ROWS
        pltpu.sync_copy(x_hbm.at[pl.ds(r0, _CH_ROWS)], xb)
        pltpu.sync_copy(y_hbm.at[pl.ds(r0, _CH_ROWS)], yb)
        carry = chunk(xb, yb, carry)

    t_acc, f_acc, c_acc = carry
    accv[pl.ds(0, 16)] = t_acc
    accv[pl.ds(16, 16)] = f_acc
    accv[pl.ds(32, 16)] = c_acc
    pltpu.sync_copy(accv, out_hbm.at[wid])


def _sc_partials(x, y):
    mesh = plsc.VectorSubcoreMesh(core_axis_name="c", subcore_axis_name="s")
    return pl.kernel(
        _sc_body,
        mesh=mesh,
        out_type=jax.ShapeDtypeStruct((_NW, 48), jnp.float32),
        scratch_types=[
            pltpu.VMEM((_CH_ROWS, _COLS), jnp.float32),
            pltpu.VMEM((_CH_ROWS, _COLS), jnp.float32),
            pltpu.VMEM((_CH_ROWS, _COLS), jnp.float32),
            pltpu.VMEM((_CH_ROWS, _COLS), jnp.float32),
            pltpu.VMEM((48,), jnp.float32),
        ],
    )(x, y)


def _tc_body(x_ref, y_ref, acc_ref):
    i = pl.program_id(0)

    @pl.when(i == 0)
    def _init():
        acc_ref[0] = 0.0
        acc_ref[1] = 0.0
        acc_ref[2] = 0.0

    x = x_ref[...]
    y = y_ref[...]
    per = jnp.maximum(x, 0.0) - x * y + jnp.log1p(jnp.exp(-jnp.abs(x)))
    acc_ref[0] += jnp.sum(per)
    acc_ref[1] += jnp.sum(per * y)
    acc_ref[2] += jnp.sum(y)


def _tc_partials(x, y):
    return pl.pallas_call(
        _tc_body,
        grid=(_TC_ROWS // _TC_BLK,),
        in_specs=[
            pl.BlockSpec((_TC_BLK, _COLS), lambda i: (i, 0)),
            pl.BlockSpec((_TC_BLK, _COLS), lambda i: (i, 0)),
        ],
        out_specs=pl.BlockSpec(memory_space=pltpu.SMEM),
        out_shape=jax.ShapeDtypeStruct((3,), jnp.float32),
    )(x, y)


def kernel(output, target):
    x = output.reshape(_ROWS, _COLS)
    y = target.reshape(_ROWS, _COLS)
    ps = _sc_partials(x, y).reshape(_NW, 3, 16).sum(axis=(0, 2))
    if _TC_ROWS:
        ps = ps + _tc_partials(x, y)
    tot, fg, cnt = ps[0], ps[1], ps[2]
    bg_cnt = jnp.maximum(float(_N) - cnt, 1.0)
    fg_cnt = jnp.maximum(cnt, 1.0)
    return (tot - fg) / bg_cnt + fg / fg_cnt


# speedup vs baseline: 1.7642x; 1.7642x over previous
import jax
import jax.numpy as jnp
from jax import lax
from jax.experimental import pallas as pl
from jax.experimental.pallas import tpu as pltpu
from jax.experimental.pallas import tpu_sc as plsc

_ROWS = 4096
_COLS = 512
_N = _ROWS * _COLS

_TC_ROWS = 3072
_SC_ROWS = _ROWS - _TC_ROWS

_NW = 32
_ROWS_W = _SC_ROWS // _NW
_CH_ROWS = 16
_NCH = _ROWS_W // _CH_ROWS
_UNROLL = 8
_VSTEPS = _CH_ROWS * (_COLS // 16) // _UNROLL

_TC_BLK = 512

_C = (
    6.944574454159635e-05,
    0.996261948233795,
    -0.46644243862757173,
    0.21866548366223043,
    -0.055459313742085355,
)


def _sc_body(x_hbm, y_hbm, out_hbm, xb0, yb0, xb1, yb1, accv, sem0, sem1):
    wid = lax.axis_index("s") * 2 + lax.axis_index("c")
    base_row = _TC_ROWS + wid * _ROWS_W
    zero = jnp.zeros((16,), jnp.float32)
    bufs = ((xb0, yb0, sem0), (xb1, yb1, sem1))

    def start(ci):
        xb, yb, sm = bufs[ci & 1]
        r0 = base_row + ci * _CH_ROWS
        hx = pltpu.async_copy(x_hbm.at[pl.ds(r0, _CH_ROWS)], xb, sm)
        hy = pltpu.async_copy(y_hbm.at[pl.ds(r0, _CH_ROWS)], yb, sm)
        return hx, hy

    def chunk(xb, yb, carry):
        def vstep(vi, carry2):
            t, f, c = carry2
            r = vi // 4
            cb = (vi % 4) * 128
            for u in range(_UNROLL):
                x = xb[r, pl.ds(cb + u * 16, 16)]
                y = yb[r, pl.ds(cb + u * 16, 16)]
                xy = x * y
                w = jnp.exp(jnp.minimum(x, -x))
                l1p = ((((_C[4] * w + _C[3]) * w + _C[2]) * w + _C[1]) * w
                       + _C[0])
                per = jnp.maximum(x, 0.0) - xy + l1p
                t = t + per
                f = f + per * y
                c = c + y
            return (t, f, c)

        return lax.fori_loop(0, _VSTEPS, vstep, carry)

    handles = {0: start(0)}
    carry = (zero, zero, zero)
    for ci in range(_NCH):
        if ci + 1 < _NCH:
            handles[ci + 1] = start(ci + 1)
        hx, hy = handles.pop(ci)
        hx.wait()
        hy.wait()
        xb, yb, _ = bufs[ci & 1]
        carry = chunk(xb, yb, carry)

    t_acc, f_acc, c_acc = carry
    accv[pl.ds(0, 16)] = t_acc
    accv[pl.ds(16, 16)] = f_acc
    accv[pl.ds(32, 16)] = c_acc
    pltpu.sync_copy(accv, out_hbm.at[wid])


def _sc_partials(x, y):
    mesh = plsc.VectorSubcoreMesh(core_axis_name="c", subcore_axis_name="s")
    return pl.kernel(
        _sc_body,
        mesh=mesh,
        out_type=jax.ShapeDtypeStruct((_NW, 48), jnp.float32),
        scratch_types=[
            pltpu.VMEM((_CH_ROWS, _COLS), jnp.float32),
            pltpu.VMEM((_CH_ROWS, _COLS), jnp.float32),
            pltpu.VMEM((_CH_ROWS, _COLS), jnp.float32),
            pltpu.VMEM((_CH_ROWS, _COLS), jnp.float32),
            pltpu.VMEM((48,), jnp.float32),
            pltpu.SemaphoreType.DMA,
            pltpu.SemaphoreType.DMA,
        ],
    )(x, y)


def _tc_body(x_ref, y_ref, acc_ref):
    i = pl.program_id(0)

    @pl.when(i == 0)
    def _init():
        acc_ref[0] = 0.0
        acc_ref[1] = 0.0
        acc_ref[2] = 0.0

    x = x_ref[...]
    y = y_ref[...]
    per = jnp.maximum(x, 0.0) - x * y + jnp.log1p(jnp.exp(-jnp.abs(x)))
    acc_ref[0] += jnp.sum(per)
    acc_ref[1] += jnp.sum(per * y)
    acc_ref[2] += jnp.sum(y)


def _tc_partials(x, y):
    return pl.pallas_call(
        _tc_body,
        grid=(_TC_ROWS // _TC_BLK,),
        in_specs=[
            pl.BlockSpec((_TC_BLK, _COLS), lambda i: (i, 0)),
            pl.BlockSpec((_TC_BLK, _COLS), lambda i: (i, 0)),
        ],
        out_specs=pl.BlockSpec(memory_space=pltpu.SMEM),
        out_shape=jax.ShapeDtypeStruct((3,), jnp.float32),
    )(x, y)


def kernel(output, target):
    x = output.reshape(_ROWS, _COLS)
    y = target.reshape(_ROWS, _COLS)
    ps = _sc_partials(x, y).reshape(_NW, 3, 16).sum(axis=(0, 2))
    if _TC_ROWS:
        ps = ps + _tc_partials(x, y)
    tot, fg, cnt = ps[0], ps[1], ps[2]
    bg_cnt = jnp.maximum(float(_N) - cnt, 1.0)
    fg_cnt = jnp.maximum(cnt, 1.0)
    return (tot - fg) / bg_cnt + fg / fg_cnt


# speedup vs baseline: 3.3150x; 1.8791x over previous
import jax
import jax.numpy as jnp
from jax.experimental import pallas as pl
from jax.experimental.pallas import tpu as pltpu

_ROWS = 4096
_COLS = 512
_BLOCK_ROWS = 512
_N_BLOCKS = _ROWS // _BLOCK_ROWS
_N_TOTAL = float(_ROWS * _COLS)

_C = (
    2.2159764900830936e-07,
    0.9999702432977317,
    -0.49933394898194267,
    0.32751171370201704,
    -0.22396689943036463,
    0.13198966240066795,
    -0.05326747773448861,
    0.01024382863145101,
)


def _body(x_ref, y_ref, loss_ref, acc_ref):
    i = pl.program_id(0)

    @pl.when(i == 0)
    def _init():
        acc_ref[0] = 0.0
        acc_ref[1] = 0.0
        acc_ref[2] = 0.0

    x = x_ref[...]
    y = y_ref[...]
    w = jnp.exp(-jnp.abs(x))
    l1p = _C[7]
    for c in (_C[6], _C[5], _C[4], _C[3], _C[2], _C[1], _C[0]):
        l1p = l1p * w + c
    per = jnp.maximum(x, 0.0) - x * y + l1p
    acc_ref[0] += jnp.sum(per)
    acc_ref[1] += jnp.sum(per * y)
    acc_ref[2] += jnp.sum(y)

    @pl.when(i == _N_BLOCKS - 1)
    def _fin():
        tot = acc_ref[0]
        fg = acc_ref[1]
        cnt = acc_ref[2]
        bg_cnt = jnp.maximum(_N_TOTAL - cnt, 1.0)
        fg_cnt = jnp.maximum(cnt, 1.0)
        loss_ref[0, 0] = (tot - fg) / bg_cnt + fg / fg_cnt


def kernel(output, target):
    x = output.reshape(_ROWS, _COLS)
    y = target.reshape(_ROWS, _COLS)
    loss = pl.pallas_call(
        _body,
        grid=(_N_BLOCKS,),
        in_specs=[
            pl.BlockSpec((_BLOCK_ROWS, _COLS), lambda i: (i, 0)),
            pl.BlockSpec((_BLOCK_ROWS, _COLS), lambda i: (i, 0)),
        ],
        out_specs=pl.BlockSpec(memory_space=pltpu.SMEM),
        out_shape=jax.ShapeDtypeStruct((1, 1), jnp.float32),
        scratch_shapes=[pltpu.SMEM((3,), jnp.float32)],
    )(x, y)
    return loss[0, 0]
